# baseline (speedup 1.0000x reference)
import functools

import jax
import jax.numpy as jnp
from jax import lax
from jax.experimental import pallas as pl
from jax.experimental.pallas import tpu as pltpu
from jax.experimental.pallas import tpu_sc as plsc

G_ = 512
P_ = 20
B_ = 4096
W_ = P_ + 2 * G_
NC_ = 2
NS_ = 16
NW_ = NC_ * NS_
BPW_ = B_ // NW_
CB_ = 32
NCH_ = BPW_ // CB_
LN_ = 16
GV_ = G_ // LN_
PP_ = 24


def _sigmoid16(x):
    return 1.0 / (1.0 + jnp.exp(-x))


def _softplus16(x):
    m = jnp.maximum(x, 0.0)
    t = jnp.exp(-jnp.abs(x))
    u = t / (2.0 + t)
    u2 = u * u
    l = 2.0 * u * (1.0 + u2 * (1.0 / 3 + u2 * (1.0 / 5 + u2 * (
        1.0 / 7 + u2 * (1.0 / 9 + u2 * (1.0 / 11))))))
    return m + l


def _sc_guide_kernel(conc_hbm, zl_hbm, zsl_hbm, idx_hbm, out_hbm,
                     idx_v, cbuf, lbuf, sbuf, outbuf, sem):
    wid = lax.axis_index("s") * NC_ + lax.axis_index("c")
    base = wid * BPW_

    for k in range(NCH_):
        row0 = base + k * CB_
        idx_row = idx_v.at[k]
        pltpu.sync_copy(idx_hbm.at[pl.ds(row0, CB_)], idx_row)

        cp_c = pltpu.async_copy(conc_hbm.at[idx_row], cbuf, sem)
        cp_l = pltpu.async_copy(zl_hbm.at[idx_row], lbuf, sem)
        cp_s = pltpu.async_copy(zsl_hbm.at[idx_row], sbuf, sem)
        cp_c.wait()
        cp_l.wait()
        cp_s.wait()

        def asm_body(i, carry):
            r = i >> 5
            c = (i & 31) * LN_
            outbuf[r, pl.ds(P_ + c, LN_)] = lbuf[r, pl.ds(c, LN_)]
            x = sbuf[r, pl.ds(c, LN_)]
            outbuf[r, pl.ds(P_ + G_ + c, LN_)] = _sigmoid16(x) * 2.0 + 0.01
            return carry

        lax.fori_loop(0, CB_ * GV_, asm_body, 0, unroll=4)

        def conc_body(r, carry):
            x0 = cbuf[r, pl.ds(0, LN_)]
            outbuf[r, pl.ds(0, LN_)] = _softplus16(x0) + 0.1
            x1 = cbuf[r, pl.ds(P_ - LN_, LN_)]
            outbuf[r, pl.ds(P_ - LN_, LN_)] = _softplus16(x1) + 0.1
            return carry

        lax.fori_loop(0, CB_, conc_body, 0, unroll=2)

        pltpu.sync_copy(outbuf, out_hbm.at[pl.ds(row0, CB_)])


@jax.jit
def _guide_sc(conc, zl, zsl, idx):
    mesh = plsc.VectorSubcoreMesh(core_axis_name="c", subcore_axis_name="s")
    run = functools.partial(
        pl.kernel,
        mesh=mesh,
        out_type=jax.ShapeDtypeStruct((B_, W_), jnp.float32),
        scratch_types=[
            pltpu.VMEM((NCH_, CB_), jnp.int32),
            pltpu.VMEM((CB_, PP_), jnp.float32),
            pltpu.VMEM((CB_, G_), jnp.float32),
            pltpu.VMEM((CB_, G_), jnp.float32),
            pltpu.VMEM((CB_, W_), jnp.float32),
            pltpu.SemaphoreType.DMA,
        ],
        compiler_params=pltpu.CompilerParams(use_tc_tiling_on_sc=False),
    )(_sc_guide_kernel)
    return run(conc, zl, zsl, idx)


def kernel(program_concentration, z_loc, z_scale_logit, cell_indices):
    idx = cell_indices.astype(jnp.int32)
    conc = jnp.pad(program_concentration, ((0, 0), (0, PP_ - P_)))
    return _guide_sc(conc, z_loc, z_scale_logit, idx)


# speedup vs baseline: 4.5032x; 4.5032x over previous
import functools

import jax
import jax.numpy as jnp
from jax import lax
from jax.experimental import pallas as pl
from jax.experimental.pallas import tpu as pltpu
from jax.experimental.pallas import tpu_sc as plsc

G_ = 512
P_ = 20
B_ = 4096
W_ = P_ + 2 * G_
NC_ = 2
NS_ = 16
NW_ = NC_ * NS_
BPW_ = B_ // NW_
CB_ = 32
NCH_ = BPW_ // CB_
RB_ = 512
PP_ = 128


def _sc_gather_kernel(conc_hbm, zl_hbm, zsl_hbm, idx_hbm,
                      cg_hbm, lg_hbm, sg_hbm,
                      idx_v, cbuf, lbuf, sbuf, sem):
    wid = lax.axis_index("s") * NC_ + lax.axis_index("c")
    base = wid * BPW_

    pltpu.sync_copy(idx_hbm.at[pl.ds(base, BPW_)], idx_v)

    for k in range(NCH_):
        row0 = base + k * CB_
        idx_row = idx_v.at[pl.ds(k * CB_, CB_)]
        cp_c = pltpu.async_copy(conc_hbm.at[idx_row], cbuf, sem)
        cp_l = pltpu.async_copy(zl_hbm.at[idx_row], lbuf, sem)
        cp_s = pltpu.async_copy(zsl_hbm.at[idx_row], sbuf, sem)
        cp_c.wait()
        cp_l.wait()
        cp_s.wait()
        pltpu.sync_copy(cbuf, cg_hbm.at[pl.ds(row0, CB_)])
        pltpu.sync_copy(lbuf, lg_hbm.at[pl.ds(row0, CB_)])
        pltpu.sync_copy(sbuf, sg_hbm.at[pl.ds(row0, CB_)])


@jax.jit
def _gather_sc(conc, zl, zsl, idx):
    mesh = plsc.VectorSubcoreMesh(core_axis_name="c", subcore_axis_name="s")
    run = functools.partial(
        pl.kernel,
        mesh=mesh,
        out_type=(
            jax.ShapeDtypeStruct((B_, PP_), jnp.float32),
            jax.ShapeDtypeStruct((B_, G_), jnp.float32),
            jax.ShapeDtypeStruct((B_, G_), jnp.float32),
        ),
        scratch_types=[
            pltpu.VMEM((BPW_,), jnp.int32),
            pltpu.VMEM((CB_, PP_), jnp.float32),
            pltpu.VMEM((CB_, G_), jnp.float32),
            pltpu.VMEM((CB_, G_), jnp.float32),
            pltpu.SemaphoreType.DMA,
        ],
    )(_sc_gather_kernel)
    return run(conc, zl, zsl, idx)


def _tc_finish_kernel(cg_ref, lg_ref, sg_ref, out_ref):
    conc = jax.nn.softplus(cg_ref[:, :P_]) + 0.1
    zs = jax.nn.sigmoid(sg_ref[...]) * 2.0 + 0.01
    out_ref[...] = jnp.concatenate([conc, lg_ref[...], zs], axis=1)


@jax.jit
def _finish_tc(cg, lg, sg):
    return pl.pallas_call(
        _tc_finish_kernel,
        grid=(B_ // RB_,),
        in_specs=[
            pl.BlockSpec((RB_, PP_), lambda i: (i, 0)),
            pl.BlockSpec((RB_, G_), lambda i: (i, 0)),
            pl.BlockSpec((RB_, G_), lambda i: (i, 0)),
        ],
        out_specs=pl.BlockSpec((RB_, W_), lambda i: (i, 0)),
        out_shape=jax.ShapeDtypeStruct((B_, W_), jnp.float32),
    )(cg, lg, sg)


def kernel(program_concentration, z_loc, z_scale_logit, cell_indices):
    idx = cell_indices.astype(jnp.int32)
    conc = jnp.pad(program_concentration, ((0, 0), (0, PP_ - P_)))
    cg, lg, sg = _gather_sc(conc, z_loc, z_scale_logit, idx)
    return _finish_tc(cg, lg, sg)
